# Initial kernel scaffold; baseline (speedup 1.0000x reference)
import jax
import jax.numpy as jnp
from jax.experimental import pallas as pl


def kernel(x, edge_index, wi, wh):
    raise NotImplementedError("write your pallas kernel here")


# speedup vs baseline: 3.0133x; 3.0133x over previous
import functools

import jax
import jax.numpy as jnp
from jax import lax
from jax.experimental import pallas as pl
from jax.experimental.pallas import tpu as pltpu
from jax.experimental.pallas import tpu_sc as plsc

LEAKY = 0.2
GATHER_WINDOW = 128
ROW_BLOCK = 1000


def _gather_edge_rows(x, idx_pad):
    P = idx_pad.shape[1]
    D = x.shape[1]
    mesh = plsc.VectorSubcoreMesh(core_axis_name="core",
                                  subcore_axis_name="subcore")
    row_ty = jax.ShapeDtypeStruct((P, D), x.dtype)

    @functools.partial(pl.kernel, out_type=(row_ty, row_ty), mesh=mesh)
    def k(x_hbm, i_hbm, src_hbm, dst_hbm):
        def body(si_vmem, di_vmem, so_vmem, do_vmem):
            pltpu.sync_copy(x_hbm.at[si_vmem.at[0]], so_vmem)
            pltpu.sync_copy(x_hbm.at[di_vmem.at[0]], do_vmem)

        pltpu.emit_pipeline(
            body,
            grid=(P // GATHER_WINDOW,),
            in_specs=[
                pl.BlockSpec((1, GATHER_WINDOW), lambda i: (0, i)),
                pl.BlockSpec((1, GATHER_WINDOW), lambda i: (1, i)),
            ],
            out_specs=[
                pl.BlockSpec((GATHER_WINDOW, D), lambda i: (i, 0)),
                pl.BlockSpec((GATHER_WINDOW, D), lambda i: (i, 0)),
            ],
            core_axis_name=("core", "subcore"),
            dimension_semantics=(pltpu.PARALLEL,),
        )(i_hbm, i_hbm, src_hbm, dst_hbm)

    return k(x, idx_pad)


def _sort_lanes(v):
    n = v.shape[-1]
    lane = lax.broadcasted_iota(jnp.int32, v.shape, len(v.shape) - 1)
    k = 2
    while k <= n:
        s = k // 2
        while s >= 1:
            lower = (lane & s) == 0
            vr = pltpu.roll(v, n - s, 1)
            vl = pltpu.roll(v, s, 1)
            partner = jnp.where(lower, vr, vl)
            keepmin = ((lane & k) == 0) == lower
            v = jnp.where(keepmin, jnp.minimum(v, partner),
                          jnp.maximum(v, partner))
            s //= 2
        k *= 2
    return v


def _tc_body(x_ref, s_ref, d_ref, wi_ref, wh_ref, o_ref):
    wi = wi_ref[...]
    f32 = jnp.float32
    x1 = jnp.dot(x_ref[...], wi, preferred_element_type=f32)
    v = jnp.dot(s_ref[...] - d_ref[...], wi, preferred_element_type=f32)
    v = _sort_lanes(v)
    ones = jnp.ones((wh_ref.shape[1], 1), f32)
    wsum = jnp.dot(wh_ref[...], ones, preferred_element_type=f32)
    s = jnp.dot(v, wsum, preferred_element_type=f32)
    o_ref[...] = (1.0 - LEAKY) * x1 + LEAKY * jnp.tanh(x1 + s)


def kernel(x, edge_index, wi, wh):
    N, D = x.shape
    P = pl.cdiv(N, GATHER_WINDOW) * GATHER_WINDOW
    idx_pad = jnp.pad(edge_index, ((0, 0), (0, P - N)))
    src_rows, dst_rows = _gather_edge_rows(x, idx_pad)

    grid = (pl.cdiv(N, ROW_BLOCK),)
    blk = lambda i: (i, 0)
    zero = lambda i: (0, 0)
    out = pl.pallas_call(
        _tc_body,
        grid=grid,
        in_specs=[
            pl.BlockSpec((ROW_BLOCK, D), blk),
            pl.BlockSpec((ROW_BLOCK, D), blk),
            pl.BlockSpec((ROW_BLOCK, D), blk),
            pl.BlockSpec((D, D), zero),
            pl.BlockSpec((D, D), zero),
        ],
        out_specs=pl.BlockSpec((ROW_BLOCK, D), blk),
        out_shape=jax.ShapeDtypeStruct((N, D), x.dtype),
    )(x, src_rows, dst_rows, wi, wh)
    return out
